# baseline (speedup 1.0000x reference)
import jax
import jax.numpy as jnp
from jax.experimental import pallas as pl
from jax.experimental.pallas import tpu as pltpu

_FEAT = 128
_Q = 100000
_B = 256
_SCALE = 30.0
_MARGIN = 0.35
_W = 2048
_NBF = 48
_WT = _Q - _NBF * _W
_K = 4
_S = 2

_DN = (((1,), (0,)), ((), ()))
_DT = (((1,), (1,)), ((), ()))


def _nrm(x):
    n = jnp.sqrt(jnp.sum(x * x, axis=1, keepdims=True))
    return x / jnp.maximum(n, 1e-12)


def _diag_m(val):
    r = jax.lax.broadcasted_iota(jnp.int32, (_B, _B), 0)
    c = jax.lax.broadcasted_iota(jnp.int32, (_B, _B), 1)
    return jnp.where(r == c, jnp.float32(val), jnp.float32(0.0))


def _body(np_ref, vg_ref, vp_ref, ng_ref, vq_ref, nq_ref, vqf_hbm, nqf_hbm,
          o1_hbm, o2_hbm, nvq_hbm, nnq_hbm,
          npn_b, vpn_b, vgn_b, ngn_b, vgt, ngt,
          o1_buf, o2_buf, pq1_buf, pq2_buf,
          vq_t, nq_t, o1_t, o2_t,
          st_sem, tl_sem, ts_sem, tq_sem):
    i = pl.program_id(0)
    s = jax.lax.rem(i, _K)

    streams = ((o1_buf, o1_hbm, _B, 0), (o2_buf, o2_hbm, _B, 1),
               (pq1_buf, nvq_hbm, _FEAT, 2), (pq2_buf, nnq_hbm, _FEAT, 3))

    def st_copies(blk, slot):
        cps = []
        for buf, hbm, rows, op in streams:
            rs = rows // _S
            for t in range(_S):
                cps.append(pltpu.make_async_copy(
                    buf.at[slot, pl.ds(t * rs, rs), :],
                    hbm.at[pl.ds(t * rs, rs), pl.ds(blk * _W, _W)],
                    st_sem.at[slot, op, t]))
        return cps

    def tail_ld_copies():
        return [pltpu.make_async_copy(
            hbm.at[:, pl.ds(_NBF * _W, _WT)], buf, tl_sem.at[op])
            for op, (hbm, buf) in enumerate(((vqf_hbm, vq_t), (nqf_hbm, nq_t)))]

    def tail_st_copies():
        cps = []
        rs = _B // _S
        for op, (buf, hbm) in enumerate(((o1_t, o1_hbm), (o2_t, o2_hbm))):
            for t in range(_S):
                cps.append(pltpu.make_async_copy(
                    buf.at[pl.ds(t * rs, rs), :],
                    hbm.at[pl.ds(t * rs, rs), pl.ds(_NBF * _W, _WT)],
                    ts_sem.at[op, t]))
        return cps

    def tail_q_copies():
        return [pltpu.make_async_copy(
            buf, hbm.at[:, pl.ds(_NBF * _W, _WT)], tq_sem.at[op])
            for op, (buf, hbm) in enumerate(((vq_t, nvq_hbm), (nq_t, nnq_hbm)))]

    @pl.when(i == 0)
    def _prologue():
        npn_b[...] = (_SCALE * _nrm(np_ref[...])).astype(jnp.bfloat16)
        vpn_b[...] = (_SCALE * _nrm(vp_ref[...])).astype(jnp.bfloat16)
        vgn = _nrm(vg_ref[...])
        ngn = _nrm(ng_ref[...])
        vgn_b[...] = vgn.astype(jnp.bfloat16)
        ngn_b[...] = ngn.astype(jnp.bfloat16)
        vgt[...] = vgn.T
        ngt[...] = ngn.T
        for c in tail_ld_copies():
            c.start()

    @pl.when(i >= _K)
    def _clear():
        for c in st_copies(i - _K, s):
            c.wait()

    c1 = jax.lax.dot_general(npn_b[...], vq_ref[...].astype(jnp.bfloat16),
                             _DN, preferred_element_type=jnp.float32)
    c2 = jax.lax.dot_general(vpn_b[...], nq_ref[...].astype(jnp.bfloat16),
                             _DN, preferred_element_type=jnp.float32)
    o1_buf[s, :, :] = jnp.clip(c1, -_SCALE, _SCALE)
    o2_buf[s, :, :] = jnp.clip(c2, -_SCALE, _SCALE)
    pq1_buf[s, :, :] = vq_ref[...]
    pq2_buf[s, :, :] = nq_ref[...]

    @pl.when(i == 0)
    def _head():
        m = _diag_m(_MARGIN * _SCALE)
        g1 = jax.lax.dot_general(npn_b[...], vgn_b[...], _DT,
                                 preferred_element_type=jnp.float32)
        g2 = jax.lax.dot_general(vpn_b[...], ngn_b[...], _DT,
                                 preferred_element_type=jnp.float32)
        o1_buf[0, :, 0:_B] = jnp.clip(g1, -_SCALE, _SCALE) - m
        o2_buf[0, :, 0:_B] = jnp.clip(g2, -_SCALE, _SCALE) - m
        pq1_buf[0, :, 0:_B] = vgt[...]
        pq2_buf[0, :, 0:_B] = ngt[...]

    for c in st_copies(i, s):
        c.start()

    @pl.when(i == _NBF - 1)
    def _tail_and_drain():
        for c in tail_ld_copies():
            c.wait()
        t1 = jax.lax.dot_general(npn_b[...], vq_t[...].astype(jnp.bfloat16),
                                 _DN, preferred_element_type=jnp.float32)
        t2 = jax.lax.dot_general(vpn_b[...], nq_t[...].astype(jnp.bfloat16),
                                 _DN, preferred_element_type=jnp.float32)
        o1_t[...] = jnp.clip(t1, -_SCALE, _SCALE)
        o2_t[...] = jnp.clip(t2, -_SCALE, _SCALE)
        for c in tail_st_copies():
            c.start()
        for c in tail_q_copies():
            c.start()
        for j in range(_NBF - _K, _NBF):
            for c in st_copies(j, j % _K):
                c.wait()
        for c in tail_st_copies():
            c.wait()
        for c in tail_q_copies():
            c.wait()


def kernel(nir_p, vis_g, vis_p, nir_g, cur_ids, vis_queue, nir_queue):
    f32 = jnp.float32
    small = pl.BlockSpec((_B, _FEAT), lambda j: (0, 0))
    colq = pl.BlockSpec((_FEAT, _W), lambda j: (0, j))
    hbm = pl.BlockSpec(memory_space=pltpu.MemorySpace.HBM)
    o1, o2, nvq, nnq = pl.pallas_call(
        _body,
        grid=(_NBF,),
        in_specs=[small, small, small, small, colq, colq, hbm, hbm],
        out_specs=(hbm, hbm, hbm, hbm),
        out_shape=(
            jax.ShapeDtypeStruct((_B, _Q), f32),
            jax.ShapeDtypeStruct((_B, _Q), f32),
            jax.ShapeDtypeStruct((_FEAT, _Q), f32),
            jax.ShapeDtypeStruct((_FEAT, _Q), f32),
        ),
        scratch_shapes=[
            pltpu.VMEM((_B, _FEAT), jnp.bfloat16),
            pltpu.VMEM((_B, _FEAT), jnp.bfloat16),
            pltpu.VMEM((_B, _FEAT), jnp.bfloat16),
            pltpu.VMEM((_B, _FEAT), jnp.bfloat16),
            pltpu.VMEM((_FEAT, _B), f32),
            pltpu.VMEM((_FEAT, _B), f32),
            pltpu.VMEM((_K, _B, _W), f32),
            pltpu.VMEM((_K, _B, _W), f32),
            pltpu.VMEM((_K, _FEAT, _W), f32),
            pltpu.VMEM((_K, _FEAT, _W), f32),
            pltpu.VMEM((_FEAT, _WT), f32),
            pltpu.VMEM((_FEAT, _WT), f32),
            pltpu.VMEM((_B, _WT), f32),
            pltpu.VMEM((_B, _WT), f32),
            pltpu.SemaphoreType.DMA((_K, 4, _S)),
            pltpu.SemaphoreType.DMA((2,)),
            pltpu.SemaphoreType.DMA((2, _S)),
            pltpu.SemaphoreType.DMA((2,)),
        ],
    )(nir_p, vis_g, vis_p, nir_g, vis_queue, nir_queue, vis_queue, nir_queue)
    label = jnp.arange(_B, dtype=jnp.int32)
    return (o1, o2, label, nvq, nnq)
